# Initial kernel scaffold; baseline (speedup 1.0000x reference)
import jax
import jax.numpy as jnp
from jax.experimental import pallas as pl


def kernel(x, embeddings):
    raise NotImplementedError("write your pallas kernel here")


# speedup vs baseline: 1.8813x; 1.8813x over previous
import functools

import jax
import jax.numpy as jnp
from jax import lax
from jax.experimental import pallas as pl
from jax.experimental.pallas import tpu as pltpu
from jax.experimental.pallas import tpu_sc as plsc

LATENT = 64
CODES = 1024
B = 16384
ROWS_PER_BLOCK = 1024

NC = 2
NS = 16
NW = NC * NS
BPW = B // NW
CHUNK = 128
NCHUNK = BPW // CHUNK


def _argmin_body(x_ref, e_ref, idx_ref):
    xb = x_ref[...]
    em = e_ref[...]
    sim = jnp.dot(xb, em, preferred_element_type=jnp.float32)
    e_sq = jnp.sum(em * em, axis=0, keepdims=True)
    dist = e_sq - 2.0 * sim
    minval = jnp.min(dist, axis=1, keepdims=True)
    cols = lax.broadcasted_iota(jnp.int32, dist.shape, 1)
    idx_ref[...] = jnp.min(
        jnp.where(dist == minval, cols, CODES), axis=1, keepdims=True
    )


def _sc_gather_body(table_hbm, idx_hbm, out_hbm, idx_v, rows_v, sem):
    wid = lax.axis_index("s") * NC + lax.axis_index("c")
    chunk_base = wid * NCHUNK
    row_base = wid * BPW
    pltpu.sync_copy(idx_hbm.at[pl.ds(chunk_base, NCHUNK)], idx_v)
    copies = [
        pltpu.async_copy(
            table_hbm.at[idx_v.at[j]],
            rows_v.at[pl.ds(j * CHUNK, CHUNK)],
            sem,
        )
        for j in range(NCHUNK)
    ]
    for cp in copies:
        cp.wait()
    pltpu.sync_copy(rows_v, out_hbm.at[pl.ds(row_base, BPW)])


def kernel(x, embeddings):
    flat = x.reshape(B, LATENT)
    idx = pl.pallas_call(
        _argmin_body,
        grid=(B // ROWS_PER_BLOCK,),
        in_specs=[
            pl.BlockSpec((ROWS_PER_BLOCK, LATENT), lambda i: (i, 0)),
            pl.BlockSpec((LATENT, CODES), lambda i: (0, 0)),
        ],
        out_specs=pl.BlockSpec((ROWS_PER_BLOCK, 1), lambda i: (i, 0)),
        out_shape=jax.ShapeDtypeStruct((B, 1), jnp.int32),
    )(flat, embeddings)

    table = embeddings.T
    idx2d = idx.reshape(B // CHUNK, CHUNK)

    gather = pl.kernel(
        _sc_gather_body,
        mesh=plsc.VectorSubcoreMesh(core_axis_name="c", subcore_axis_name="s"),
        out_type=jax.ShapeDtypeStruct((B, LATENT), jnp.float32),
        scratch_types=[
            pltpu.VMEM((NCHUNK, CHUNK), jnp.int32),
            pltpu.VMEM((BPW, LATENT), jnp.float32),
            pltpu.SemaphoreType.DMA,
        ],
        compiler_params=pltpu.CompilerParams(use_tc_tiling_on_sc=False),
    )
    quantized = gather(table, idx2d)
    return quantized.reshape(x.shape)
